# baseline (speedup 1.0000x reference)
import functools

import jax
import jax.numpy as jnp
import numpy as np
from jax import lax
from jax.experimental import pallas as pl
from jax.experimental.pallas import tpu as pltpu
from jax.experimental.pallas import tpu_sc as plsc

D = 768
H = 12
DH = 64
E = 8
FF = 3072
CAP = 320
TQ = 512


def _sc_embed(table, idx):
    info = plsc.get_sparse_core_info()
    nw = info.num_cores * info.num_subcores
    tt = idx.shape[0]
    d = table.shape[1]
    b_per_w = tt // nw
    mesh = plsc.VectorSubcoreMesh(core_axis_name="c", subcore_axis_name="s")

    @functools.partial(
        pl.kernel,
        mesh=mesh,
        out_type=jax.ShapeDtypeStruct((tt, d), jnp.float32),
        scratch_types=[
            pltpu.VMEM((b_per_w,), jnp.int32),
            pltpu.VMEM((b_per_w, d), jnp.float32),
            pltpu.SemaphoreType.DMA,
        ],
    )
    def k(table_hbm, idx_hbm, out_hbm, idx_v, rows_v, sem):
        wid = lax.axis_index("s") * info.num_cores + lax.axis_index("c")
        base = wid * b_per_w
        pltpu.sync_copy(idx_hbm.at[pl.ds(base, b_per_w)], idx_v)
        pltpu.async_copy(table_hbm.at[idx_v], rows_v, sem).wait()
        pltpu.sync_copy(rows_v, out_hbm.at[pl.ds(base, b_per_w)])

    return k(table, idx)


def _ln(x, s, b):
    mu = jnp.mean(x, axis=-1, keepdims=True)
    var = jnp.mean((x - mu) ** 2, axis=-1, keepdims=True)
    return (x - mu) / jnp.sqrt(var + 1e-5) * s + b


def _lnqkv_body(h_ref, s_ref, b_ref, wq_ref, wk_ref, wv_ref, q_ref, k_ref, v_ref):
    hn = _ln(h_ref[...], s_ref[...], b_ref[...])
    for w_ref, o_ref in ((wq_ref, q_ref), (wk_ref, k_ref), (wv_ref, v_ref)):
        p = jnp.dot(hn, w_ref[...], preferred_element_type=jnp.float32)
        for hh in range(H):
            o_ref[hh] = p[:, hh * DH:(hh + 1) * DH]


def _attn_body(tok_ref, q_ref, k_ref, v_ref, o_ref):
    qh = q_ref[0]
    kh = k_ref[0]
    vh = v_ref[0]
    s = lax.dot_general(qh, kh, (((1,), (1,)), ((), ())),
                        preferred_element_type=jnp.float32) * (1.0 / np.sqrt(DH))
    s = jnp.where(tok_ref[...] != 0, s, -1e9)
    p = jax.nn.softmax(s, axis=-1)
    o_ref[0] = jnp.dot(p, vh, preferred_element_type=jnp.float32)


def _wo_res_body(o_ref, wo_ref, h_ref, out_ref):
    t = h_ref.shape[0]
    acc = jnp.zeros((t, D), jnp.float32)
    for hh in range(H):
        acc = acc + jnp.dot(o_ref[hh], wo_ref[hh * DH:(hh + 1) * DH, :],
                            preferred_element_type=jnp.float32)
    out_ref[...] = h_ref[...] + acc


def _route_body(h2_ref, s_ref, b_ref, rw_ref, hn_ref, slot_ref, gate_ref):
    t = h2_ref.shape[0]
    hn = _ln(h2_ref[...], s_ref[...], b_ref[...])
    hn_ref[...] = hn
    logits = jnp.dot(hn, rw_ref[...], preferred_element_type=jnp.float32)
    probs = jax.nn.softmax(logits, axis=-1)
    gate = jnp.max(probs, axis=-1, keepdims=True)
    gate_ref[...] = gate
    iota_e = lax.broadcasted_iota(jnp.int32, (t, E), 1)
    eidx = jnp.min(jnp.where(probs == gate, iota_e, E), axis=-1, keepdims=True)
    onehot = (iota_e == eidx).astype(jnp.float32)
    pos = onehot
    k = 1
    while k < t:
        pos = pos + jnp.concatenate(
            [jnp.zeros((k, E), jnp.float32), pos[:-k]], axis=0)
        k *= 2
    slot_ref[...] = (pos * onehot).astype(jnp.int32)


NF = 2
FFB = FF // NF


def _expert_body(hn_ref, slot_ref, gate_ref, h2_ref, w1_ref, b1_ref, w2_ref,
                 b2_ref, out_ref, xe_ref, ye_ref, acc_ref):
    e = pl.program_id(0)
    f = pl.program_id(1)
    t = hn_ref.shape[0]
    iota_e = lax.broadcasted_iota(jnp.int32, (t, E), 1)
    sel = (iota_e == e).astype(jnp.int32)
    slot = jnp.sum(slot_ref[...] * sel, axis=-1, keepdims=True) - 1
    iota_c = lax.broadcasted_iota(jnp.int32, (t, CAP), 1)
    disp = (iota_c == slot).astype(jnp.float32)

    @pl.when(f == 0)
    def _():
        xe_ref[...] = lax.dot_general(disp, hn_ref[...], (((0,), (0,)), ((), ())),
                                      preferred_element_type=jnp.float32)

    h1 = jnp.maximum(
        jnp.dot(xe_ref[...], w1_ref[0], preferred_element_type=jnp.float32)
        + b1_ref[0], 0.0)
    part = jnp.dot(h1, w2_ref[0], preferred_element_type=jnp.float32)

    @pl.when(f == 0)
    def _():
        ye_ref[...] = part

    @pl.when(f != 0)
    def _():
        ye_ref[...] = ye_ref[...] + part

    @pl.when(f == NF - 1)
    def _():
        ye = ye_ref[...] + b2_ref[0]
        contrib = jnp.dot(disp, ye, preferred_element_type=jnp.float32)

        @pl.when(e == 0)
        def _():
            acc_ref[...] = contrib

        @pl.when(e != 0)
        def _():
            acc_ref[...] = acc_ref[...] + contrib

        @pl.when(e == E - 1)
        def _():
            out_ref[...] = h2_ref[...] + gate_ref[...] * acc_ref[...]


def _vocab_body(h_ref, emb_ref, out_ref):
    out_ref[...] = lax.dot_general(h_ref[...], emb_ref[...],
                                   (((1,), (1,)), ((), ())),
                                   preferred_element_type=jnp.float32)


def kernel(x, embedding, Wq, Wk, Wv, Wo, ln1_s, ln1_b, ln2_s, ln2_b,
           router_w, w1, b1, w2, b2):
    b, t = x.shape
    tt = b * t
    vocab = embedding.shape[0]
    tok = x.reshape(tt).astype(jnp.int32)
    tok_row = tok.reshape(1, tt)
    nlayers = Wq.shape[0]

    h = _sc_embed(embedding, tok)

    lnqkv = pl.pallas_call(
        _lnqkv_body,
        out_shape=[jax.ShapeDtypeStruct((H, tt, DH), jnp.float32)] * 3,
    )
    attn = pl.pallas_call(
        _attn_body,
        grid=(H, tt // TQ),
        in_specs=[
            pl.BlockSpec((1, tt), lambda hh, i: (0, 0)),
            pl.BlockSpec((1, TQ, DH), lambda hh, i: (hh, i, 0)),
            pl.BlockSpec((1, tt, DH), lambda hh, i: (hh, 0, 0)),
            pl.BlockSpec((1, tt, DH), lambda hh, i: (hh, 0, 0)),
        ],
        out_specs=pl.BlockSpec((1, TQ, DH), lambda hh, i: (hh, i, 0)),
        out_shape=jax.ShapeDtypeStruct((H, tt, DH), jnp.float32),
    )
    wo_res = pl.pallas_call(
        _wo_res_body,
        out_shape=jax.ShapeDtypeStruct((tt, D), jnp.float32),
    )
    route = pl.pallas_call(
        _route_body,
        out_shape=[
            jax.ShapeDtypeStruct((tt, D), jnp.float32),
            jax.ShapeDtypeStruct((tt, E), jnp.int32),
            jax.ShapeDtypeStruct((tt, 1), jnp.float32),
        ],
    )
    expert = pl.pallas_call(
        _expert_body,
        grid=(E, NF),
        in_specs=[
            pl.BlockSpec((tt, D), lambda e, f: (0, 0)),
            pl.BlockSpec((tt, E), lambda e, f: (0, 0)),
            pl.BlockSpec((tt, 1), lambda e, f: (0, 0)),
            pl.BlockSpec((tt, D), lambda e, f: (0, 0)),
            pl.BlockSpec((1, D, FFB), lambda e, f: (e, 0, f)),
            pl.BlockSpec((1, 1, FFB), lambda e, f: (e, 0, f)),
            pl.BlockSpec((1, FFB, D), lambda e, f: (e, f, 0)),
            pl.BlockSpec((1, 1, D), lambda e, f: (e, 0, 0)),
        ],
        out_specs=pl.BlockSpec((tt, D), lambda e, f: (0, 0)),
        out_shape=jax.ShapeDtypeStruct((tt, D), jnp.float32),
        scratch_shapes=[
            pltpu.VMEM((CAP, D), jnp.float32),
            pltpu.VMEM((CAP, D), jnp.float32),
            pltpu.VMEM((tt, D), jnp.float32),
        ],
    )

    for l in range(nlayers):
        q, k, v = lnqkv(h, ln1_s[l].reshape(1, D), ln1_b[l].reshape(1, D),
                        Wq[l], Wk[l], Wv[l])
        o = attn(tok_row, q, k, v)
        h = wo_res(o, Wo[l], h)
        hn2, slotmat, gate = route(h, ln2_s[l].reshape(1, D),
                                   ln2_b[l].reshape(1, D), router_w[l])
        h = expert(hn2, slotmat, gate, h, w1[l], b1[l].reshape(E, 1, FF),
                   w2[l], b2[l].reshape(E, 1, D))

    vb = 1280
    out = pl.pallas_call(
        _vocab_body,
        grid=(vocab // vb,),
        in_specs=[
            pl.BlockSpec((tt, D), lambda i: (0, 0)),
            pl.BlockSpec((vb, D), lambda i: (i, 0)),
        ],
        out_specs=pl.BlockSpec((tt, vb), lambda i: (0, i)),
        out_shape=jax.ShapeDtypeStruct((tt, vocab), jnp.float32),
    )(h, embedding)
    return out.reshape(b, t, vocab)


# speedup vs baseline: 1.1405x; 1.1405x over previous
import functools

import jax
import jax.numpy as jnp
import numpy as np
from jax import lax
from jax.experimental import pallas as pl
from jax.experimental.pallas import tpu as pltpu
from jax.experimental.pallas import tpu_sc as plsc

D = 768
H = 12
DH = 64
E = 8
FF = 3072
CAP = 320
TQ = 512
NF = 2
FFB = FF // NF


def _sc_embed(table, idx):
    info = plsc.get_sparse_core_info()
    nw = info.num_cores * info.num_subcores
    tt = idx.shape[0]
    d = table.shape[1]
    b_per_w = tt // nw
    mesh = plsc.VectorSubcoreMesh(core_axis_name="c", subcore_axis_name="s")

    @functools.partial(
        pl.kernel,
        mesh=mesh,
        out_type=jax.ShapeDtypeStruct((tt, d), jnp.float32),
        scratch_types=[
            pltpu.VMEM((b_per_w,), jnp.int32),
            pltpu.VMEM((b_per_w, d), jnp.float32),
            pltpu.SemaphoreType.DMA,
        ],
    )
    def k(table_hbm, idx_hbm, out_hbm, idx_v, rows_v, sem):
        wid = lax.axis_index("s") * info.num_cores + lax.axis_index("c")
        base = wid * b_per_w
        pltpu.sync_copy(idx_hbm.at[pl.ds(base, b_per_w)], idx_v)
        pltpu.async_copy(table_hbm.at[idx_v], rows_v, sem).wait()
        pltpu.sync_copy(rows_v, out_hbm.at[pl.ds(base, b_per_w)])

    return k(table, idx)


def _ln(x, s, b):
    mu = jnp.mean(x, axis=-1, keepdims=True)
    var = jnp.mean((x - mu) ** 2, axis=-1, keepdims=True)
    return (x - mu) / jnp.sqrt(var + 1e-5) * s + b


def _attnblock_body(h_ref, tok_ref, s_ref, b_ref, wq_ref, wk_ref, wv_ref,
                    wo_ref, out_ref, kf_ref, vf_ref):
    i = pl.program_id(0)

    @pl.when(i == 0)
    def _():
        hn = _ln(h_ref[...], s_ref[...], b_ref[...])
        kf_ref[...] = jnp.dot(hn, wk_ref[...], preferred_element_type=jnp.float32)
        vf_ref[...] = jnp.dot(hn, wv_ref[...], preferred_element_type=jnp.float32)

    hblk = h_ref[pl.ds(i * TQ, TQ), :]
    hq = _ln(hblk, s_ref[...], b_ref[...])
    q = jnp.dot(hq, wq_ref[...], preferred_element_type=jnp.float32)
    valid = tok_ref[...] != 0
    ohs = []
    for hh in range(H):
        sl = slice(hh * DH, (hh + 1) * DH)
        s = lax.dot_general(q[:, sl], kf_ref[:, sl], (((1,), (1,)), ((), ())),
                            preferred_element_type=jnp.float32) * (1.0 / np.sqrt(DH))
        s = jnp.where(valid, s, -1e9)
        p = jax.nn.softmax(s, axis=-1)
        ohs.append(jnp.dot(p, vf_ref[:, sl], preferred_element_type=jnp.float32))
    o = jnp.concatenate(ohs, axis=1)
    out_ref[...] = hblk + jnp.dot(o, wo_ref[...], preferred_element_type=jnp.float32)


def _moe_body(h2_ref, s_ref, b_ref, rw_ref, w1_ref, b1_ref, w2_ref, b2_ref,
              out_ref, hn_ref, slot_ref, gate_ref, xe_ref, ye_ref, acc_ref):
    e = pl.program_id(0)
    f = pl.program_id(1)
    t = h2_ref.shape[0]

    @pl.when(jnp.logical_and(e == 0, f == 0))
    def _():
        hn = _ln(h2_ref[...], s_ref[...], b_ref[...])
        hn_ref[...] = hn
        logits = jnp.dot(hn, rw_ref[...], preferred_element_type=jnp.float32)
        probs = jax.nn.softmax(logits, axis=-1)
        gate = jnp.max(probs, axis=-1, keepdims=True)
        gate_ref[...] = gate
        iota_e = lax.broadcasted_iota(jnp.int32, (t, E), 1)
        eidx = jnp.min(jnp.where(probs == gate, iota_e, E), axis=-1, keepdims=True)
        onehot = (iota_e == eidx).astype(jnp.float32)
        pos = onehot
        k = 1
        while k < t:
            pos = pos + jnp.concatenate(
                [jnp.zeros((k, E), jnp.float32), pos[:-k]], axis=0)
            k *= 2
        slot_ref[...] = (pos * onehot).astype(jnp.int32)

    iota_e = lax.broadcasted_iota(jnp.int32, (t, E), 1)
    sel = (iota_e == e).astype(jnp.int32)
    slot = jnp.sum(slot_ref[...] * sel, axis=-1, keepdims=True) - 1
    iota_c = lax.broadcasted_iota(jnp.int32, (t, CAP), 1)
    disp = (iota_c == slot).astype(jnp.float32)

    @pl.when(f == 0)
    def _():
        xe_ref[...] = lax.dot_general(disp, hn_ref[...], (((0,), (0,)), ((), ())),
                                      preferred_element_type=jnp.float32)

    h1 = jnp.maximum(
        jnp.dot(xe_ref[...], w1_ref[0], preferred_element_type=jnp.float32)
        + b1_ref[0], 0.0)
    part = jnp.dot(h1, w2_ref[0], preferred_element_type=jnp.float32)

    @pl.when(f == 0)
    def _():
        ye_ref[...] = part

    @pl.when(f != 0)
    def _():
        ye_ref[...] = ye_ref[...] + part

    @pl.when(f == NF - 1)
    def _():
        ye = ye_ref[...] + b2_ref[0]
        contrib = jnp.dot(disp, ye, preferred_element_type=jnp.float32)

        @pl.when(e == 0)
        def _():
            acc_ref[...] = contrib

        @pl.when(e != 0)
        def _():
            acc_ref[...] = acc_ref[...] + contrib

        @pl.when(e == E - 1)
        def _():
            out_ref[...] = h2_ref[...] + gate_ref[...] * acc_ref[...]


def _vocab_body(h_ref, emb_ref, out_ref, hb_ref):
    @pl.when(pl.program_id(0) == 0)
    def _():
        hb_ref[...] = h_ref[...].astype(jnp.bfloat16)

    eb = emb_ref[...].astype(jnp.bfloat16)
    out_ref[...] = lax.dot_general(hb_ref[...], eb, (((1,), (1,)), ((), ())),
                                   preferred_element_type=jnp.float32)


def kernel(x, embedding, Wq, Wk, Wv, Wo, ln1_s, ln1_b, ln2_s, ln2_b,
           router_w, w1, b1, w2, b2):
    b, t = x.shape
    tt = b * t
    vocab = embedding.shape[0]
    tok = x.reshape(tt).astype(jnp.int32)
    tok_row = tok.reshape(1, tt)
    nlayers = Wq.shape[0]

    h = _sc_embed(embedding, tok)

    attnblock = pl.pallas_call(
        _attnblock_body,
        grid=(tt // TQ,),
        in_specs=[
            pl.BlockSpec((tt, D), lambda i: (0, 0)),
            pl.BlockSpec((1, tt), lambda i: (0, 0)),
            pl.BlockSpec((1, D), lambda i: (0, 0)),
            pl.BlockSpec((1, D), lambda i: (0, 0)),
            pl.BlockSpec((D, D), lambda i: (0, 0)),
            pl.BlockSpec((D, D), lambda i: (0, 0)),
            pl.BlockSpec((D, D), lambda i: (0, 0)),
            pl.BlockSpec((D, D), lambda i: (0, 0)),
        ],
        out_specs=pl.BlockSpec((TQ, D), lambda i: (i, 0)),
        out_shape=jax.ShapeDtypeStruct((tt, D), jnp.float32),
        scratch_shapes=[
            pltpu.VMEM((tt, D), jnp.float32),
            pltpu.VMEM((tt, D), jnp.float32),
        ],
    )
    moe = pl.pallas_call(
        _moe_body,
        grid=(E, NF),
        in_specs=[
            pl.BlockSpec((tt, D), lambda e, f: (0, 0)),
            pl.BlockSpec((1, D), lambda e, f: (0, 0)),
            pl.BlockSpec((1, D), lambda e, f: (0, 0)),
            pl.BlockSpec((D, E), lambda e, f: (0, 0)),
            pl.BlockSpec((1, D, FFB), lambda e, f: (e, 0, f)),
            pl.BlockSpec((1, 1, FFB), lambda e, f: (e, 0, f)),
            pl.BlockSpec((1, FFB, D), lambda e, f: (e, f, 0)),
            pl.BlockSpec((1, 1, D), lambda e, f: (e, 0, 0)),
        ],
        out_specs=pl.BlockSpec((tt, D), lambda e, f: (0, 0)),
        out_shape=jax.ShapeDtypeStruct((tt, D), jnp.float32),
        scratch_shapes=[
            pltpu.VMEM((tt, D), jnp.float32),
            pltpu.VMEM((tt, E), jnp.int32),
            pltpu.VMEM((tt, 1), jnp.float32),
            pltpu.VMEM((CAP, D), jnp.float32),
            pltpu.VMEM((CAP, D), jnp.float32),
            pltpu.VMEM((tt, D), jnp.float32),
        ],
    )

    for l in range(nlayers):
        h = attnblock(h, tok_row, ln1_s[l].reshape(1, D), ln1_b[l].reshape(1, D),
                      Wq[l], Wk[l], Wv[l], Wo[l])
        h = moe(h, ln2_s[l].reshape(1, D), ln2_b[l].reshape(1, D),
                router_w[l], w1[l], b1[l].reshape(E, 1, FF),
                w2[l], b2[l].reshape(E, 1, D))

    vb = 1280
    out = pl.pallas_call(
        _vocab_body,
        grid=(vocab // vb,),
        in_specs=[
            pl.BlockSpec((tt, D), lambda i: (0, 0)),
            pl.BlockSpec((vb, D), lambda i: (i, 0)),
        ],
        out_specs=pl.BlockSpec((tt, vb), lambda i: (0, i)),
        out_shape=jax.ShapeDtypeStruct((tt, vocab), jnp.float32),
        scratch_shapes=[pltpu.VMEM((tt, D), jnp.bfloat16)],
    )(h, embedding)
    return out.reshape(b, t, vocab)


# speedup vs baseline: 1.3292x; 1.1654x over previous
import functools

import jax
import jax.numpy as jnp
import numpy as np
from jax import lax
from jax.experimental import pallas as pl
from jax.experimental.pallas import tpu as pltpu
from jax.experimental.pallas import tpu_sc as plsc

D = 768
H = 12
DH = 64
E = 8
FF = 3072
CAP = 320
TQ = 512
NF = 2
FFB = FF // NF


def _sc_embed(table, idx):
    info = plsc.get_sparse_core_info()
    nw = info.num_cores * info.num_subcores
    tt = idx.shape[0]
    d = table.shape[1]
    b_per_w = tt // nw
    mesh = plsc.VectorSubcoreMesh(core_axis_name="c", subcore_axis_name="s")

    @functools.partial(
        pl.kernel,
        mesh=mesh,
        out_type=jax.ShapeDtypeStruct((tt, d), jnp.float32),
        scratch_types=[
            pltpu.VMEM((b_per_w,), jnp.int32),
            pltpu.VMEM((b_per_w, d), jnp.float32),
            pltpu.SemaphoreType.DMA,
        ],
    )
    def k(table_hbm, idx_hbm, out_hbm, idx_v, rows_v, sem):
        wid = lax.axis_index("s") * info.num_cores + lax.axis_index("c")
        base = wid * b_per_w
        pltpu.sync_copy(idx_hbm.at[pl.ds(base, b_per_w)], idx_v)
        pltpu.async_copy(table_hbm.at[idx_v], rows_v, sem).wait()
        pltpu.sync_copy(rows_v, out_hbm.at[pl.ds(base, b_per_w)])

    return k(table, idx)


def _ln(x, s, b):
    mu = jnp.mean(x, axis=-1, keepdims=True)
    var = jnp.mean((x - mu) ** 2, axis=-1, keepdims=True)
    return (x - mu) / jnp.sqrt(var + 1e-5) * s + b


def _attnblock_body(h_ref, tok_ref, s_ref, b_ref, wq_ref, wk_ref, wv_ref,
                    wo_ref, out_ref, kf_ref, vf_ref):
    i = pl.program_id(0)

    @pl.when(i == 0)
    def _():
        hn = _ln(h_ref[...], s_ref[...], b_ref[...])
        kf_ref[...] = jnp.dot(
            hn, wk_ref[...], preferred_element_type=jnp.float32
        ).astype(jnp.bfloat16)
        vf_ref[...] = jnp.dot(
            hn, wv_ref[...], preferred_element_type=jnp.float32
        ).astype(jnp.bfloat16)

    hblk = h_ref[pl.ds(i * TQ, TQ), :]
    hq = _ln(hblk, s_ref[...], b_ref[...])
    q = (jnp.dot(hq, wq_ref[...], preferred_element_type=jnp.float32)
         * (1.0 / np.sqrt(DH))).astype(jnp.bfloat16)
    valid = tok_ref[...] != 0
    ohs = []
    for hh in range(H):
        sl = slice(hh * DH, (hh + 1) * DH)
        s = lax.dot_general(q[:, sl], kf_ref[:, sl], (((1,), (1,)), ((), ())),
                            preferred_element_type=jnp.float32)
        ex = jnp.where(valid, jnp.exp(s), 0.0)
        r = 1.0 / jnp.sum(ex, axis=-1, keepdims=True)
        av = jnp.dot(ex.astype(jnp.bfloat16), vf_ref[:, sl],
                     preferred_element_type=jnp.float32)
        ohs.append(av * r)
    o = jnp.concatenate(ohs, axis=1).astype(jnp.bfloat16)
    out_ref[...] = hblk + jnp.dot(o, wo_ref[...].astype(jnp.bfloat16),
                                  preferred_element_type=jnp.float32)


def _moe_body(h2_ref, s_ref, b_ref, rw_ref, w1_ref, b1_ref, w2_ref, b2_ref,
              out_ref, hn_ref, slot_ref, gate_ref, xe_ref, ye_ref, acc_ref):
    e = pl.program_id(0)
    f = pl.program_id(1)
    t = h2_ref.shape[0]

    @pl.when(jnp.logical_and(e == 0, f == 0))
    def _():
        hn = _ln(h2_ref[...], s_ref[...], b_ref[...])
        hn_ref[...] = hn
        logits = jnp.dot(hn, rw_ref[...], preferred_element_type=jnp.float32)
        probs = jax.nn.softmax(logits, axis=-1)
        gate = jnp.max(probs, axis=-1, keepdims=True)
        gate_ref[...] = gate
        iota_e = lax.broadcasted_iota(jnp.int32, (t, E), 1)
        eidx = jnp.min(jnp.where(probs == gate, iota_e, E), axis=-1, keepdims=True)
        onehot = (iota_e == eidx).astype(jnp.float32)
        pos = onehot
        k = 1
        while k < t:
            pos = pos + jnp.concatenate(
                [jnp.zeros((k, E), jnp.float32), pos[:-k]], axis=0)
            k *= 2
        slot_ref[...] = (pos * onehot).astype(jnp.int32)

    iota_e = lax.broadcasted_iota(jnp.int32, (t, E), 1)
    sel = (iota_e == e).astype(jnp.int32)
    slot = jnp.sum(slot_ref[...] * sel, axis=-1, keepdims=True) - 1
    iota_c = lax.broadcasted_iota(jnp.int32, (t, CAP), 1)
    disp = (iota_c == slot).astype(jnp.float32)

    dispb = disp.astype(jnp.bfloat16)

    @pl.when(f == 0)
    def _():
        xe_ref[...] = lax.dot_general(
            dispb, hn_ref[...].astype(jnp.bfloat16), (((0,), (0,)), ((), ())),
            preferred_element_type=jnp.float32).astype(jnp.bfloat16)

    h1 = jnp.maximum(
        jnp.dot(xe_ref[...], w1_ref[0].astype(jnp.bfloat16),
                preferred_element_type=jnp.float32)
        + b1_ref[0], 0.0).astype(jnp.bfloat16)
    part = jnp.dot(h1, w2_ref[0].astype(jnp.bfloat16),
                   preferred_element_type=jnp.float32)

    @pl.when(f == 0)
    def _():
        ye_ref[...] = part

    @pl.when(f != 0)
    def _():
        ye_ref[...] = ye_ref[...] + part

    @pl.when(f == NF - 1)
    def _():
        ye = (ye_ref[...] + b2_ref[0]).astype(jnp.bfloat16)
        contrib = jnp.dot(dispb, ye, preferred_element_type=jnp.float32)

        @pl.when(e == 0)
        def _():
            acc_ref[...] = contrib

        @pl.when(e != 0)
        def _():
            acc_ref[...] = acc_ref[...] + contrib

        @pl.when(e == E - 1)
        def _():
            out_ref[...] = h2_ref[...] + gate_ref[...] * acc_ref[...]


def _vocab_body(h_ref, emb_ref, out_ref, hb_ref):
    @pl.when(pl.program_id(0) == 0)
    def _():
        hb_ref[...] = h_ref[...].astype(jnp.bfloat16)

    eb = emb_ref[...].astype(jnp.bfloat16)
    out_ref[...] = lax.dot_general(hb_ref[...], eb, (((1,), (1,)), ((), ())),
                                   preferred_element_type=jnp.float32)


def kernel(x, embedding, Wq, Wk, Wv, Wo, ln1_s, ln1_b, ln2_s, ln2_b,
           router_w, w1, b1, w2, b2):
    b, t = x.shape
    tt = b * t
    vocab = embedding.shape[0]
    tok = x.reshape(tt).astype(jnp.int32)
    tok_row = tok.reshape(1, tt)
    nlayers = Wq.shape[0]

    h = _sc_embed(embedding, tok)

    attnblock = pl.pallas_call(
        _attnblock_body,
        grid=(tt // TQ,),
        in_specs=[
            pl.BlockSpec((tt, D), lambda i: (0, 0)),
            pl.BlockSpec((1, tt), lambda i: (0, 0)),
            pl.BlockSpec((1, D), lambda i: (0, 0)),
            pl.BlockSpec((1, D), lambda i: (0, 0)),
            pl.BlockSpec((D, D), lambda i: (0, 0)),
            pl.BlockSpec((D, D), lambda i: (0, 0)),
            pl.BlockSpec((D, D), lambda i: (0, 0)),
            pl.BlockSpec((D, D), lambda i: (0, 0)),
        ],
        out_specs=pl.BlockSpec((TQ, D), lambda i: (i, 0)),
        out_shape=jax.ShapeDtypeStruct((tt, D), jnp.float32),
        scratch_shapes=[
            pltpu.VMEM((tt, D), jnp.bfloat16),
            pltpu.VMEM((tt, D), jnp.bfloat16),
        ],
    )
    moe = pl.pallas_call(
        _moe_body,
        grid=(E, NF),
        in_specs=[
            pl.BlockSpec((tt, D), lambda e, f: (0, 0)),
            pl.BlockSpec((1, D), lambda e, f: (0, 0)),
            pl.BlockSpec((1, D), lambda e, f: (0, 0)),
            pl.BlockSpec((D, E), lambda e, f: (0, 0)),
            pl.BlockSpec((1, D, FFB), lambda e, f: (e, 0, f)),
            pl.BlockSpec((1, 1, FFB), lambda e, f: (e, 0, f)),
            pl.BlockSpec((1, FFB, D), lambda e, f: (e, f, 0)),
            pl.BlockSpec((1, 1, D), lambda e, f: (e, 0, 0)),
        ],
        out_specs=pl.BlockSpec((tt, D), lambda e, f: (0, 0)),
        out_shape=jax.ShapeDtypeStruct((tt, D), jnp.float32),
        scratch_shapes=[
            pltpu.VMEM((tt, D), jnp.float32),
            pltpu.VMEM((tt, E), jnp.int32),
            pltpu.VMEM((tt, 1), jnp.float32),
            pltpu.VMEM((CAP, D), jnp.bfloat16),
            pltpu.VMEM((CAP, D), jnp.float32),
            pltpu.VMEM((tt, D), jnp.float32),
        ],
    )

    for l in range(nlayers):
        h = attnblock(h, tok_row, ln1_s[l].reshape(1, D), ln1_b[l].reshape(1, D),
                      Wq[l], Wk[l], Wv[l], Wo[l])
        h = moe(h, ln2_s[l].reshape(1, D), ln2_b[l].reshape(1, D),
                router_w[l], w1[l], b1[l].reshape(E, 1, FF),
                w2[l], b2[l].reshape(E, 1, D))

    vb = 1280
    out = pl.pallas_call(
        _vocab_body,
        grid=(vocab // vb,),
        in_specs=[
            pl.BlockSpec((tt, D), lambda i: (0, 0)),
            pl.BlockSpec((vb, D), lambda i: (i, 0)),
        ],
        out_specs=pl.BlockSpec((tt, vb), lambda i: (0, i)),
        out_shape=jax.ShapeDtypeStruct((tt, vocab), jnp.float32),
        scratch_shapes=[pltpu.VMEM((tt, D), jnp.bfloat16)],
    )(h, embedding)
    return out.reshape(b, t, vocab)


# speedup vs baseline: 1.3904x; 1.0461x over previous
import functools

import jax
import jax.numpy as jnp
import numpy as np
from jax import lax
from jax.experimental import pallas as pl
from jax.experimental.pallas import tpu as pltpu
from jax.experimental.pallas import tpu_sc as plsc

D = 768
H = 12
DH = 64
E = 8
FF = 3072
CAP = 320
TQ = 512
NF = 2
FFB = FF // NF


def _sc_embed(table, idx):
    info = plsc.get_sparse_core_info()
    nw = info.num_cores * info.num_subcores
    tt = idx.shape[0]
    d = table.shape[1]
    b_per_w = tt // nw
    mesh = plsc.VectorSubcoreMesh(core_axis_name="c", subcore_axis_name="s")

    @functools.partial(
        pl.kernel,
        mesh=mesh,
        out_type=jax.ShapeDtypeStruct((tt, d), jnp.float32),
        scratch_types=[
            pltpu.VMEM((b_per_w,), jnp.int32),
            pltpu.VMEM((b_per_w, d), jnp.float32),
            pltpu.SemaphoreType.DMA,
        ],
    )
    def k(table_hbm, idx_hbm, out_hbm, idx_v, rows_v, sem):
        wid = lax.axis_index("s") * info.num_cores + lax.axis_index("c")
        base = wid * b_per_w
        pltpu.sync_copy(idx_hbm.at[pl.ds(base, b_per_w)], idx_v)
        pltpu.async_copy(table_hbm.at[idx_v], rows_v, sem).wait()
        pltpu.sync_copy(rows_v, out_hbm.at[pl.ds(base, b_per_w)])

    return k(table, idx)


def _sc_scatter(rows, idx, nrows_out):
    info = plsc.get_sparse_core_info()
    nw = info.num_cores * info.num_subcores
    tt = idx.shape[0]
    d = rows.shape[1]
    b_per_w = tt // nw
    mesh = plsc.VectorSubcoreMesh(core_axis_name="c", subcore_axis_name="s")

    @functools.partial(
        pl.kernel,
        mesh=mesh,
        out_type=jax.ShapeDtypeStruct((nrows_out, d), jnp.float32),
        scratch_types=[
            pltpu.VMEM((b_per_w,), jnp.int32),
            pltpu.VMEM((b_per_w, d), jnp.float32),
            pltpu.SemaphoreType.DMA,
        ],
    )
    def k(rows_hbm, idx_hbm, out_hbm, idx_v, rows_v, sem):
        wid = lax.axis_index("s") * info.num_cores + lax.axis_index("c")
        base = wid * b_per_w
        pltpu.sync_copy(idx_hbm.at[pl.ds(base, b_per_w)], idx_v)
        pltpu.sync_copy(rows_hbm.at[pl.ds(base, b_per_w)], rows_v)
        pltpu.async_copy(rows_v, out_hbm.at[idx_v], sem).wait()

    return k(rows, idx)


def _ln(x, s, b):
    mu = jnp.mean(x, axis=-1, keepdims=True)
    var = jnp.mean((x - mu) ** 2, axis=-1, keepdims=True)
    return (x - mu) / jnp.sqrt(var + 1e-5) * s + b


def _attn_compute(i, h_ref, tok_ref, s_ref, b_ref, wq_ref, wk_ref, wv_ref,
                  wo_ref, out_ref, kfx_ref, vfx_ref):
    t = h_ref.shape[0]

    @pl.when(i == 0)
    def _():
        hn = _ln(h_ref[...], s_ref[...], b_ref[...]).astype(jnp.bfloat16)
        kf = jnp.dot(hn, wk_ref[...].astype(jnp.bfloat16),
                     preferred_element_type=jnp.float32).astype(jnp.bfloat16)
        vf = jnp.dot(hn, wv_ref[...].astype(jnp.bfloat16),
                     preferred_element_type=jnp.float32).astype(jnp.bfloat16)
        mbias = jnp.where(tok_ref[...] != 0, 0.0, -40.0).astype(jnp.bfloat16)
        onec = jnp.ones((t, 1), jnp.bfloat16)
        zpad = jnp.zeros((t, 127 - DH), jnp.bfloat16)
        kparts, vparts = [], []
        for hh in range(H):
            sl = slice(hh * DH, (hh + 1) * DH)
            kparts += [kf[:, sl], mbias, zpad]
            vparts += [vf[:, sl], onec, zpad]
        kfx_ref[...] = jnp.concatenate(kparts, axis=1)
        vfx_ref[...] = jnp.concatenate(vparts, axis=1)

    hblk = h_ref[pl.ds(i * TQ, TQ), :]
    hq = _ln(hblk, s_ref[...], b_ref[...]).astype(jnp.bfloat16)
    q = (jnp.dot(hq, wq_ref[...].astype(jnp.bfloat16),
                 preferred_element_type=jnp.float32)
         * (1.0 / np.sqrt(DH))).astype(jnp.bfloat16)
    oneq = jnp.ones((TQ, 1), jnp.bfloat16)
    zq = jnp.zeros((TQ, 127 - DH), jnp.bfloat16)
    ohs = []
    for hh in range(H):
        sl = slice(hh * DH, (hh + 1) * DH)
        slx = slice(hh * 128, (hh + 1) * 128)
        qx = jnp.concatenate([q[:, sl], oneq, zq], axis=1)
        s = lax.dot_general(qx, kfx_ref[:, slx], (((1,), (1,)), ((), ())),
                            preferred_element_type=jnp.float32)
        ex = jnp.exp(s).astype(jnp.bfloat16)
        av = jnp.dot(ex, vfx_ref[:, slx],
                     preferred_element_type=jnp.float32)
        r = 1.0 / av[:, DH:DH + 1]
        ohs.append(av[:, :DH] * r)
    o = jnp.concatenate(ohs, axis=1).astype(jnp.bfloat16)
    out_ref[...] = hblk + jnp.dot(o, wo_ref[...].astype(jnp.bfloat16),
                                  preferred_element_type=jnp.float32)


def _attnblock_body(h_ref, tok_ref, s_ref, b_ref, wq_ref, wk_ref, wv_ref,
                    wo_ref, out_ref, kfx_ref, vfx_ref):
    _attn_compute(pl.program_id(0), h_ref, tok_ref, s_ref, b_ref, wq_ref,
                  wk_ref, wv_ref, wo_ref, out_ref, kfx_ref, vfx_ref)


def _attnfuse_body(h_ref, y_ref, g_ref, tok_ref, s_ref, b_ref, wq_ref,
                   wk_ref, wv_ref, wo_ref, out_ref, kfx_ref, vfx_ref, hf_ref):
    i = pl.program_id(0)

    @pl.when(i == 0)
    def _():
        g = g_ref[...]
        hf_ref[...] = h_ref[...] + jnp.where(g > 0.0, y_ref[...] * g, 0.0)

    _attn_compute(i, hf_ref, tok_ref, s_ref, b_ref, wq_ref, wk_ref, wv_ref,
                  wo_ref, out_ref, kfx_ref, vfx_ref)


def _route_body(h2_ref, s_ref, b_ref, rw_ref, hn_ref, dst_ref, gatev_ref):
    t = h2_ref.shape[0]
    hn = _ln(h2_ref[...], s_ref[...], b_ref[...])
    hn_ref[...] = hn
    logits = jnp.dot(hn, rw_ref[...], preferred_element_type=jnp.float32)
    probs = jax.nn.softmax(logits, axis=-1)
    gate = jnp.max(probs, axis=-1, keepdims=True)
    iota_e = lax.broadcasted_iota(jnp.int32, (t, E), 1)
    eidx = jnp.min(jnp.where(probs == gate, iota_e, E), axis=-1, keepdims=True)
    onehot = (iota_e == eidx).astype(jnp.float32)
    pos = onehot
    k = 1
    while k < t:
        pos = pos + jnp.concatenate(
            [jnp.zeros((k, E), jnp.float32), pos[:-k]], axis=0)
        k *= 2
    slot = jnp.sum(pos * onehot, axis=-1, keepdims=True).astype(jnp.int32)
    valid = slot <= CAP
    dst_ref[...] = jnp.where(valid, eidx * CAP + slot - 1, E * CAP)
    gatev_ref[...] = jnp.where(valid, gate, 0.0)


def _ffn_body(x_ref, w1_ref, b1_ref, w2_ref, b2_ref, out_ref, ye_ref):
    f = pl.program_id(1)
    xe = x_ref[...].astype(jnp.bfloat16)
    h1 = jnp.maximum(
        jnp.dot(xe, w1_ref[0].astype(jnp.bfloat16),
                preferred_element_type=jnp.float32)
        + b1_ref[0], 0.0).astype(jnp.bfloat16)
    part = jnp.dot(h1, w2_ref[0].astype(jnp.bfloat16),
                   preferred_element_type=jnp.float32)

    @pl.when(f == 0)
    def _():
        ye_ref[...] = part

    @pl.when(f != 0)
    def _():
        ye_ref[...] = ye_ref[...] + part

    @pl.when(f == NF - 1)
    def _():
        out_ref[...] = ye_ref[...] + b2_ref[0]


def _vocab_body(h_ref, y_ref, g_ref, emb_ref, out_ref, hb_ref):
    @pl.when(pl.program_id(0) == 0)
    def _():
        g = g_ref[...]
        hb_ref[...] = (h_ref[...]
                       + jnp.where(g > 0.0, y_ref[...] * g, 0.0)
                       ).astype(jnp.bfloat16)

    eb = emb_ref[...].astype(jnp.bfloat16)
    out_ref[...] = lax.dot_general(hb_ref[...], eb, (((1,), (1,)), ((), ())),
                                   preferred_element_type=jnp.float32)


def kernel(x, embedding, Wq, Wk, Wv, Wo, ln1_s, ln1_b, ln2_s, ln2_b,
           router_w, w1, b1, w2, b2):
    b, t = x.shape
    tt = b * t
    vocab = embedding.shape[0]
    tok = x.reshape(tt).astype(jnp.int32)
    tok_col = tok.reshape(tt, 1)
    nlayers = Wq.shape[0]

    h = _sc_embed(embedding, tok)

    attnblock = pl.pallas_call(
        _attnblock_body,
        grid=(tt // TQ,),
        in_specs=[
            pl.BlockSpec((tt, D), lambda i: (0, 0)),
            pl.BlockSpec((tt, 1), lambda i: (0, 0)),
            pl.BlockSpec((1, D), lambda i: (0, 0)),
            pl.BlockSpec((1, D), lambda i: (0, 0)),
            pl.BlockSpec((D, D), lambda i: (0, 0)),
            pl.BlockSpec((D, D), lambda i: (0, 0)),
            pl.BlockSpec((D, D), lambda i: (0, 0)),
            pl.BlockSpec((D, D), lambda i: (0, 0)),
        ],
        out_specs=pl.BlockSpec((TQ, D), lambda i: (i, 0)),
        out_shape=jax.ShapeDtypeStruct((tt, D), jnp.float32),
        scratch_shapes=[
            pltpu.VMEM((tt, H * 128), jnp.bfloat16),
            pltpu.VMEM((tt, H * 128), jnp.bfloat16),
        ],
    )
    route = pl.pallas_call(
        _route_body,
        in_specs=[
            pl.BlockSpec((tt, D), lambda: (0, 0)),
            pl.BlockSpec((1, D), lambda: (0, 0)),
            pl.BlockSpec((1, D), lambda: (0, 0)),
            pl.BlockSpec((D, E), lambda: (0, 0)),
        ],
        out_specs=[
            pl.BlockSpec((tt, D), lambda: (0, 0)),
            pl.BlockSpec((tt, 1), lambda: (0, 0)),
            pl.BlockSpec((tt, 1), lambda: (0, 0)),
        ],
        out_shape=[
            jax.ShapeDtypeStruct((tt, D), jnp.float32),
            jax.ShapeDtypeStruct((tt, 1), jnp.int32),
            jax.ShapeDtypeStruct((tt, 1), jnp.float32),
        ],
    )
    nbuf = E * CAP + CAP
    ffn = pl.pallas_call(
        _ffn_body,
        grid=(E, NF),
        in_specs=[
            pl.BlockSpec((CAP, D), lambda e, f: (e, 0)),
            pl.BlockSpec((1, D, FFB), lambda e, f: (e, 0, f)),
            pl.BlockSpec((1, 1, FFB), lambda e, f: (e, 0, f)),
            pl.BlockSpec((1, FFB, D), lambda e, f: (e, f, 0)),
            pl.BlockSpec((1, 1, D), lambda e, f: (e, 0, 0)),
        ],
        out_specs=pl.BlockSpec((CAP, D), lambda e, f: (e, 0)),
        out_shape=jax.ShapeDtypeStruct((nbuf, D), jnp.float32),
        scratch_shapes=[pltpu.VMEM((CAP, D), jnp.float32)],
    )
    attnfuse = pl.pallas_call(
        _attnfuse_body,
        grid=(tt // TQ,),
        in_specs=[
            pl.BlockSpec((tt, D), lambda i: (0, 0)),
            pl.BlockSpec((tt, D), lambda i: (0, 0)),
            pl.BlockSpec((tt, 1), lambda i: (0, 0)),
            pl.BlockSpec((tt, 1), lambda i: (0, 0)),
            pl.BlockSpec((1, D), lambda i: (0, 0)),
            pl.BlockSpec((1, D), lambda i: (0, 0)),
            pl.BlockSpec((D, D), lambda i: (0, 0)),
            pl.BlockSpec((D, D), lambda i: (0, 0)),
            pl.BlockSpec((D, D), lambda i: (0, 0)),
            pl.BlockSpec((D, D), lambda i: (0, 0)),
        ],
        out_specs=pl.BlockSpec((TQ, D), lambda i: (i, 0)),
        out_shape=jax.ShapeDtypeStruct((tt, D), jnp.float32),
        scratch_shapes=[
            pltpu.VMEM((tt, H * 128), jnp.bfloat16),
            pltpu.VMEM((tt, H * 128), jnp.bfloat16),
            pltpu.VMEM((tt, D), jnp.float32),
        ],
    )

    y_prev = None
    for l in range(nlayers):
        if l == 0:
            h = attnblock(h, tok_col, ln1_s[l].reshape(1, D),
                          ln1_b[l].reshape(1, D), Wq[l], Wk[l], Wv[l], Wo[l])
        else:
            h = attnfuse(h, y_prev, g_prev, tok_col, ln1_s[l].reshape(1, D),
                         ln1_b[l].reshape(1, D), Wq[l], Wk[l], Wv[l], Wo[l])
        hn, dst2, gatev = route(h, ln2_s[l].reshape(1, D),
                                ln2_b[l].reshape(1, D), router_w[l])
        dst = dst2.reshape(tt)
        xbuf = _sc_scatter(hn, dst, nbuf)
        ybuf = ffn(xbuf, w1[l], b1[l].reshape(E, 1, FF),
                   w2[l], b2[l].reshape(E, 1, D))
        y_prev = _sc_embed(ybuf, dst)
        g_prev = gatev

    vb = 1280
    out = pl.pallas_call(
        _vocab_body,
        grid=(vocab // vb,),
        in_specs=[
            pl.BlockSpec((tt, D), lambda i: (0, 0)),
            pl.BlockSpec((tt, D), lambda i: (0, 0)),
            pl.BlockSpec((tt, 1), lambda i: (0, 0)),
            pl.BlockSpec((vb, D), lambda i: (i, 0)),
        ],
        out_specs=pl.BlockSpec((tt, vb), lambda i: (0, i)),
        out_shape=jax.ShapeDtypeStruct((tt, vocab), jnp.float32),
        scratch_shapes=[pltpu.VMEM((tt, D), jnp.bfloat16)],
    )(h, y_prev, g_prev, embedding)
    return out.reshape(b, t, vocab)
